# baseline (speedup 1.0000x reference)
import functools

import jax
import jax.numpy as jnp
from jax import lax
from jax.experimental import pallas as pl
from jax.experimental.pallas import tpu as pltpu
from jax.experimental.pallas import tpu_sc as plsc
from jax.sharding import PartitionSpec as P

_COMMIT = 0.25
_TAU = 0.5

_TN = 512
_TK = 1024
_TN2 = 256
_TK2 = 1024

_KS1 = 42
_KS2 = 0x1BD11BF0
_TINY = 1.1754943508222875e-38


def _tf_rounds(x0, x1, rots):
    for r in rots:
        x0 = x0 + x1
        x1 = lax.shift_left(x1, jnp.uint32(r)) | lax.shift_right_logical(
            x1, jnp.uint32(32 - r))
        x1 = x1 ^ x0
    return x0, x1


def _gumbel_tile(row0, j, k_total, shape):
    ra = (13, 15, 26, 6)
    rb = (17, 29, 16, 24)
    base = (row0 * k_total + j * shape[1] + _KS1).astype(jnp.uint32)
    rr = lax.broadcasted_iota(jnp.uint32, shape, 0)
    cc = lax.broadcasted_iota(jnp.uint32, shape, 1)
    x1 = rr * jnp.uint32(k_total) + cc + base
    x0 = jnp.zeros(shape, jnp.uint32)
    x0, x1 = _tf_rounds(x0, x1, ra)
    x0, x1 = x0 + jnp.uint32(_KS1), x1 + jnp.uint32(_KS2 + 1)
    x0, x1 = _tf_rounds(x0, x1, rb)
    x0, x1 = x0 + jnp.uint32(_KS2), x1 + jnp.uint32(2)
    x0, x1 = _tf_rounds(x0, x1, ra)
    x0, x1 = x0, x1 + jnp.uint32(_KS1 + 3)
    x0, x1 = _tf_rounds(x0, x1, rb)
    x0, x1 = x0 + jnp.uint32(_KS1), x1 + jnp.uint32(_KS2 + 4)
    x0, x1 = _tf_rounds(x0, x1, ra)
    x0, x1 = x0 + jnp.uint32(_KS2), x1 + jnp.uint32(5)
    bits = x0 ^ x1
    fb = lax.shift_right_logical(bits, jnp.uint32(9)) | jnp.uint32(0x3F800000)
    fl = lax.bitcast_convert_type(fb, jnp.float32) - 1.0
    u = jnp.maximum(jnp.float32(_TINY), fl + jnp.float32(_TINY))
    return -jnp.log(-jnp.log(u))


def _dist_body(roff_ref, xn_ref, wn_ref, x_ref, w_ref, ent_ref, idx_ref,
               m_s, s_s, u_s, best_s, bidx_s, k_total, tn):
    i = pl.program_id(0)
    j = pl.program_id(1)
    nk = pl.num_programs(1)

    @pl.when(j == 0)
    def _init():
        m_s[...] = jnp.full(m_s.shape, -1e30, jnp.float32)
        s_s[...] = jnp.zeros(s_s.shape, jnp.float32)
        u_s[...] = jnp.zeros(u_s.shape, jnp.float32)
        best_s[...] = jnp.full(best_s.shape, -jnp.inf, jnp.float32)
        bidx_s[...] = jnp.zeros(bidx_s.shape, jnp.int32)

    dot = lax.dot_general(
        x_ref[...].astype(jnp.bfloat16), w_ref[...].astype(jnp.bfloat16),
        (((1,), (1,)), ((), ())), preferred_element_type=jnp.float32)
    d = (xn_ref[...] + wn_ref[...]) - 2.0 * dot
    g = _gumbel_tile(roff_ref[0] + i * tn, j, k_total, d.shape)
    sc = g - d
    tmax = jnp.max(sc, axis=1, keepdims=True)
    col = lax.broadcasted_iota(jnp.int32, sc.shape, 1) + j * _TK
    cand = jnp.min(jnp.where(sc == tmax, col, jnp.int32(2 ** 30)),
                   axis=1, keepdims=True)
    upd = tmax > best_s[...]
    best_s[...] = jnp.where(upd, tmax, best_s[...])
    bidx_s[...] = jnp.where(upd, cand, bidx_s[...])

    l = -2.0 * d
    mt = jnp.max(l, axis=1, keepdims=True)
    m_old = m_s[...]
    m_new = jnp.maximum(m_old, mt)
    scale = jnp.exp(m_old - m_new)
    t = l - m_new
    e = jnp.exp(t)
    s_t = jnp.sum(e, axis=1, keepdims=True)
    u_t = jnp.sum(t * e, axis=1, keepdims=True)
    s_old = s_s[...]
    u_s[...] = (u_s[...] + (m_old - m_new) * s_old) * scale + u_t
    s_s[...] = s_old * scale + s_t
    m_s[...] = m_new

    @pl.when(j == nk - 1)
    def _fin():
        s = s_s[...]
        ent_ref[...] = u_s[...] / s - jnp.log(s)
        idx_ref[...] = bidx_s[...]


def _dist_call(roff, xn, wn, flat, weight):
    n, ddim = flat.shape
    k = weight.shape[0]
    tn = next(t for t in (_TN, 256, 128, 64, 32, 16, 8) if n % t == 0)
    grid = (n // tn, k // _TK)
    return pl.pallas_call(
        functools.partial(_dist_body, k_total=k, tn=tn),
        grid=grid,
        in_specs=[
            pl.BlockSpec(memory_space=pltpu.SMEM),
            pl.BlockSpec((tn, 1), lambda i, j: (i, 0)),
            pl.BlockSpec((1, _TK), lambda i, j: (0, j)),
            pl.BlockSpec((tn, ddim), lambda i, j: (i, 0)),
            pl.BlockSpec((_TK, ddim), lambda i, j: (j, 0)),
        ],
        out_specs=[
            pl.BlockSpec((tn, 1), lambda i, j: (i, 0)),
            pl.BlockSpec((tn, 1), lambda i, j: (i, 0)),
        ],
        out_shape=[
            jax.ShapeDtypeStruct((n, 1), jnp.float32),
            jax.ShapeDtypeStruct((n, 1), jnp.int32),
        ],
        scratch_shapes=[
            pltpu.VMEM((tn, 1), jnp.float32),
            pltpu.VMEM((tn, 1), jnp.float32),
            pltpu.VMEM((tn, 1), jnp.float32),
            pltpu.VMEM((tn, 1), jnp.float32),
            pltpu.VMEM((tn, 1), jnp.int32),
        ],
        compiler_params=pltpu.CompilerParams(
            dimension_semantics=("arbitrary", "arbitrary")),
    )(roff, xn, wn, flat, weight)


def _onehot_body(idx_ref, ent_ref, enc_ref, cnt_ref, esum_ref,
                 cnt_s, esum_s):
    jj = pl.program_id(0)
    ii = pl.program_id(1)
    ni = pl.num_programs(1)

    idx = idx_ref[...]
    col = lax.broadcasted_iota(jnp.int32, (_TN2, _TK2), 1) + jj * _TK2
    e = (col == idx).astype(jnp.float32)
    enc_ref[...] = e

    @pl.when(ii == 0)
    def _zero_cnt():
        cnt_s[...] = jnp.zeros(cnt_s.shape, jnp.float32)

    cnt_s[...] += jnp.sum(e, axis=0, keepdims=True)

    @pl.when(jj == 0)
    def _acc_ent():
        @pl.when(ii == 0)
        def _zero_esum():
            esum_s[...] = jnp.zeros(esum_s.shape, jnp.float32)

        esum_s[...] += jnp.sum(ent_ref[...])

    @pl.when(ii == ni - 1)
    def _fin():
        cnt_ref[...] = cnt_s[...]
        esum_ref[...] = esum_s[...]


def _onehot_call(idx, ent, k):
    n = idx.shape[0]
    grid = (k // _TK2, n // _TN2)
    return pl.pallas_call(
        _onehot_body,
        grid=grid,
        in_specs=[
            pl.BlockSpec((_TN2, 1), lambda jj, ii: (ii, 0)),
            pl.BlockSpec((_TN2, 1), lambda jj, ii: (ii, 0)),
        ],
        out_specs=[
            pl.BlockSpec((_TN2, _TK2), lambda jj, ii: (ii, jj)),
            pl.BlockSpec((1, _TK2), lambda jj, ii: (0, jj)),
            pl.BlockSpec((1, 1), lambda jj, ii: (0, 0)),
        ],
        out_shape=[
            jax.ShapeDtypeStruct((n, k), jnp.float32),
            jax.ShapeDtypeStruct((1, k), jnp.float32),
            jax.ShapeDtypeStruct((1, 1), jnp.float32),
        ],
        scratch_shapes=[
            pltpu.VMEM((1, _TK2), jnp.float32),
            pltpu.VMEM((1, 1), jnp.float32),
        ],
        compiler_params=pltpu.CompilerParams(
            dimension_semantics=("arbitrary", "arbitrary")),
    )(idx, ent)


def _finalize_body(cnt_ref, esum_ref, loss_ref, perp_ref, n_total):
    avg = cnt_ref[...] / float(n_total)
    pent = jnp.sum(avg * jnp.log(avg + 1e-10), axis=1, keepdims=True)
    loss_ref[...] = _COMMIT * (esum_ref[...] / float(n_total))
    perp_ref[...] = jnp.exp(-pent)


def _finalize_call(cnt, esum, n_total):
    return pl.pallas_call(
        functools.partial(_finalize_body, n_total=n_total),
        out_shape=[
            jax.ShapeDtypeStruct((1, 1), jnp.float32),
            jax.ShapeDtypeStruct((1, 1), jnp.float32),
        ],
    )(cnt, esum)


def _sc_gather_build(n, ddim):
    try:
        info = plsc.get_sparse_core_info()
        nw = info.num_cores * info.num_subcores
        ncores = info.num_cores
    except Exception:
        nw, ncores = 32, 2
    bpw = n // nw
    mesh = plsc.VectorSubcoreMesh(core_axis_name="c", subcore_axis_name="s")

    @functools.partial(
        pl.kernel,
        out_type=jax.ShapeDtypeStruct((n, ddim), jnp.float32),
        mesh=mesh,
        scratch_types=[
            pltpu.VMEM((bpw,), jnp.int32),
            pltpu.VMEM((bpw, ddim), jnp.float32),
            pltpu.SemaphoreType.DMA,
        ],
    )
    def _sc_gather(w_hbm, idx_hbm, out_hbm, idx_v, rows_v, sem):
        wid = lax.axis_index("s") * ncores + lax.axis_index("c")
        base = wid * bpw
        pltpu.sync_copy(idx_hbm.at[pl.ds(base, bpw)], idx_v)
        pltpu.async_copy(w_hbm.at[idx_v], rows_v, sem).wait()
        pltpu.sync_copy(rows_v, out_hbm.at[pl.ds(base, bpw)])

    return _sc_gather


def _pipeline(inputs_l, weight, roff, n_total, axis_name):
    x = jnp.transpose(inputs_l, (0, 2, 3, 1))
    b, h, w, dd = x.shape
    flat = x.reshape(-1, dd)
    k = weight.shape[0]

    xn = jnp.sum(flat ** 2, axis=1, keepdims=True)
    wn = jnp.sum(weight ** 2, axis=1)[None, :]

    ent, idx = _dist_call(roff, xn, wn, flat, weight)
    quant = _sc_gather_build(flat.shape[0], dd)(weight, idx.reshape(-1))
    enc, cnt, esum = _onehot_call(idx, ent, k)

    if axis_name is not None:
        cnt = lax.psum(cnt, axis_name)
        esum = lax.psum(esum, axis_name)
    loss11, perp11 = _finalize_call(cnt, esum, n_total)

    q_l = jnp.transpose(quant.reshape(b, h, w, dd), (0, 3, 1, 2))
    return loss11, q_l, perp11, enc


def kernel(inputs, weight):
    n_total = inputs.shape[0] * inputs.shape[2] * inputs.shape[3]
    devs = jax.devices()
    if len(devs) >= 2 and inputs.shape[0] % 2 == 0:
        mesh = jax.sharding.Mesh(devs[:2], ("x",))
        nl = n_total // 2

        def _shard_fn(inputs_l, weight_r):
            off = (lax.axis_index("x") * nl).astype(jnp.int32)[None]
            return _pipeline(inputs_l, weight_r, off, n_total, "x")

        loss11, q_out, perp11, enc = jax.shard_map(
            _shard_fn,
            mesh=mesh,
            in_specs=(P("x", None, None, None), P(None, None)),
            out_specs=(P(None, None), P("x", None, None, None),
                       P(None, None), P("x", None)),
            check_vma=False,
        )(inputs, weight)
    else:
        roff = jnp.zeros((1,), jnp.int32)
        loss11, q_out, perp11, enc = _pipeline(
            inputs, weight, roff, n_total, None)
    return (loss11[0, 0], q_out, perp11[0, 0], enc)


# speedup vs baseline: 1.0837x; 1.0837x over previous
import functools

import jax
import jax.numpy as jnp
from jax import lax
from jax.experimental import pallas as pl
from jax.experimental.pallas import tpu as pltpu
from jax.experimental.pallas import tpu_sc as plsc
from jax.sharding import PartitionSpec as P

_COMMIT = 0.25
_TAU = 0.5

_TN = 512
_TK = 1024
_TN2 = 512
_TK2 = 2048

_KS1 = 42
_KS2 = 0x1BD11BF0
_TINY = 1.1754943508222875e-38


def _tf_rounds(x0, x1, rots):
    for r in rots:
        x0 = x0 + x1
        x1 = lax.shift_left(x1, jnp.uint32(r)) | lax.shift_right_logical(
            x1, jnp.uint32(32 - r))
        x1 = x1 ^ x0
    return x0, x1


def _gumbel_tile(row0, j, k_total, shape):
    ra = (13, 15, 26, 6)
    rb = (17, 29, 16, 24)
    base = (row0 * k_total + j * shape[1] + _KS1).astype(jnp.uint32)
    rr = lax.broadcasted_iota(jnp.uint32, shape, 0)
    cc = lax.broadcasted_iota(jnp.uint32, shape, 1)
    x1 = rr * jnp.uint32(k_total) + cc + base
    x0 = x1
    x1 = (lax.shift_left(x1, jnp.uint32(13))
          | lax.shift_right_logical(x1, jnp.uint32(19))) ^ x0
    x0, x1 = _tf_rounds(x0, x1, ra[1:])
    x0, x1 = x0 + jnp.uint32(_KS1), x1 + jnp.uint32(_KS2 + 1)
    x0, x1 = _tf_rounds(x0, x1, rb)
    x0, x1 = x0 + jnp.uint32(_KS2), x1 + jnp.uint32(2)
    x0, x1 = _tf_rounds(x0, x1, ra)
    x0, x1 = x0, x1 + jnp.uint32(_KS1 + 3)
    x0, x1 = _tf_rounds(x0, x1, rb)
    x0, x1 = x0 + jnp.uint32(_KS1), x1 + jnp.uint32(_KS2 + 4)
    x0, x1 = _tf_rounds(x0, x1, ra)
    x0, x1 = x0 + jnp.uint32(_KS2), x1 + jnp.uint32(5)
    bits = x0 ^ x1
    fb = lax.shift_right_logical(bits, jnp.uint32(9)) | jnp.uint32(0x3F800000)
    fl = lax.bitcast_convert_type(fb, jnp.float32) - 1.0
    u = jnp.maximum(jnp.float32(_TINY), fl + jnp.float32(_TINY))
    return -jnp.log(-jnp.log(u))


def _dist_body(roff_ref, xn_ref, wn_ref, x_ref, w_ref, ent_ref, idx_ref,
               m_s, s_s, u_s, best_s, bidx_s, k_total, tn):
    i = pl.program_id(0)
    j = pl.program_id(1)
    nk = pl.num_programs(1)

    @pl.when(j == 0)
    def _init():
        m_s[...] = jnp.full(m_s.shape, -1e30, jnp.float32)
        s_s[...] = jnp.zeros(s_s.shape, jnp.float32)
        u_s[...] = jnp.zeros(u_s.shape, jnp.float32)
        best_s[...] = jnp.full(best_s.shape, -jnp.inf, jnp.float32)
        bidx_s[...] = jnp.zeros(bidx_s.shape, jnp.int32)

    dot = lax.dot_general(
        x_ref[...].astype(jnp.bfloat16), w_ref[...].astype(jnp.bfloat16),
        (((1,), (1,)), ((), ())), preferred_element_type=jnp.float32)
    d = (xn_ref[...] + wn_ref[...]) - 2.0 * dot
    g = _gumbel_tile(roff_ref[0] + i * tn, j, k_total, d.shape)
    sc = g - d
    tmax = jnp.max(sc, axis=1, keepdims=True)
    col = lax.broadcasted_iota(jnp.int32, sc.shape, 1) + j * _TK
    cand = jnp.min(jnp.where(sc == tmax, col, jnp.int32(2 ** 30)),
                   axis=1, keepdims=True)
    upd = tmax > best_s[...]
    best_s[...] = jnp.where(upd, tmax, best_s[...])
    bidx_s[...] = jnp.where(upd, cand, bidx_s[...])

    l = -2.0 * d
    mt = jnp.max(l, axis=1, keepdims=True)
    m_old = m_s[...]
    m_new = jnp.maximum(m_old, mt)
    scale = jnp.exp(m_old - m_new)
    t = l - m_new
    e = jnp.exp(t)
    s_t = jnp.sum(e, axis=1, keepdims=True)
    u_t = jnp.sum(t * e, axis=1, keepdims=True)
    s_old = s_s[...]
    u_s[...] = (u_s[...] + (m_old - m_new) * s_old) * scale + u_t
    s_s[...] = s_old * scale + s_t
    m_s[...] = m_new

    @pl.when(j == nk - 1)
    def _fin():
        s = s_s[...]
        ent_ref[...] = u_s[...] / s - jnp.log(s)
        idx_ref[...] = bidx_s[...]


def _dist_call(roff, xn, wn, flat, weight):
    n, ddim = flat.shape
    k = weight.shape[0]
    tn = next(t for t in (_TN, 256, 128, 64, 32, 16, 8) if n % t == 0)
    grid = (n // tn, k // _TK)
    return pl.pallas_call(
        functools.partial(_dist_body, k_total=k, tn=tn),
        grid=grid,
        in_specs=[
            pl.BlockSpec(memory_space=pltpu.SMEM),
            pl.BlockSpec((tn, 1), lambda i, j: (i, 0)),
            pl.BlockSpec((1, _TK), lambda i, j: (0, j)),
            pl.BlockSpec((tn, ddim), lambda i, j: (i, 0)),
            pl.BlockSpec((_TK, ddim), lambda i, j: (j, 0)),
        ],
        out_specs=[
            pl.BlockSpec((tn, 1), lambda i, j: (i, 0)),
            pl.BlockSpec((tn, 1), lambda i, j: (i, 0)),
        ],
        out_shape=[
            jax.ShapeDtypeStruct((n, 1), jnp.float32),
            jax.ShapeDtypeStruct((n, 1), jnp.int32),
        ],
        scratch_shapes=[
            pltpu.VMEM((tn, 1), jnp.float32),
            pltpu.VMEM((tn, 1), jnp.float32),
            pltpu.VMEM((tn, 1), jnp.float32),
            pltpu.VMEM((tn, 1), jnp.float32),
            pltpu.VMEM((tn, 1), jnp.int32),
        ],
        compiler_params=pltpu.CompilerParams(
            dimension_semantics=("arbitrary", "arbitrary")),
    )(roff, xn, wn, flat, weight)


def _onehot_body(idx_ref, ent_ref, enc_ref, cnt_ref, esum_ref,
                 cnt_s, esum_s):
    jj = pl.program_id(0)
    ii = pl.program_id(1)
    ni = pl.num_programs(1)

    idx = idx_ref[...]
    col = lax.broadcasted_iota(jnp.int32, (_TN2, _TK2), 1) + jj * _TK2
    e = (col == idx).astype(jnp.float32)
    enc_ref[...] = e

    @pl.when(ii == 0)
    def _zero_cnt():
        cnt_s[...] = jnp.zeros(cnt_s.shape, jnp.float32)

    cnt_s[...] += jnp.sum(e, axis=0, keepdims=True)

    @pl.when(jj == 0)
    def _acc_ent():
        @pl.when(ii == 0)
        def _zero_esum():
            esum_s[...] = jnp.zeros(esum_s.shape, jnp.float32)

        esum_s[...] += jnp.sum(ent_ref[...])

    @pl.when(ii == ni - 1)
    def _fin():
        cnt_ref[...] = cnt_s[...]
        esum_ref[...] = esum_s[...]


def _onehot_call(idx, ent, k):
    n = idx.shape[0]
    grid = (k // _TK2, n // _TN2)
    return pl.pallas_call(
        _onehot_body,
        grid=grid,
        in_specs=[
            pl.BlockSpec((_TN2, 1), lambda jj, ii: (ii, 0)),
            pl.BlockSpec((_TN2, 1), lambda jj, ii: (ii, 0)),
        ],
        out_specs=[
            pl.BlockSpec((_TN2, _TK2), lambda jj, ii: (ii, jj)),
            pl.BlockSpec((1, _TK2), lambda jj, ii: (0, jj)),
            pl.BlockSpec((1, 1), lambda jj, ii: (0, 0)),
        ],
        out_shape=[
            jax.ShapeDtypeStruct((n, k), jnp.float32),
            jax.ShapeDtypeStruct((1, k), jnp.float32),
            jax.ShapeDtypeStruct((1, 1), jnp.float32),
        ],
        scratch_shapes=[
            pltpu.VMEM((1, _TK2), jnp.float32),
            pltpu.VMEM((1, 1), jnp.float32),
        ],
        compiler_params=pltpu.CompilerParams(
            dimension_semantics=("arbitrary", "arbitrary")),
    )(idx, ent)


def _finalize_body(cnt_ref, esum_ref, loss_ref, perp_ref, n_total):
    avg = cnt_ref[...] / float(n_total)
    pent = jnp.sum(avg * jnp.log(avg + 1e-10), axis=1, keepdims=True)
    loss_ref[...] = _COMMIT * (esum_ref[...] / float(n_total))
    perp_ref[...] = jnp.exp(-pent)


def _finalize_call(cnt, esum, n_total):
    return pl.pallas_call(
        functools.partial(_finalize_body, n_total=n_total),
        out_shape=[
            jax.ShapeDtypeStruct((1, 1), jnp.float32),
            jax.ShapeDtypeStruct((1, 1), jnp.float32),
        ],
    )(cnt, esum)


def _sc_gather_build(n, ddim):
    try:
        info = plsc.get_sparse_core_info()
        nw = info.num_cores * info.num_subcores
        ncores = info.num_cores
    except Exception:
        nw, ncores = 32, 2
    bpw = n // nw
    mesh = plsc.VectorSubcoreMesh(core_axis_name="c", subcore_axis_name="s")

    @functools.partial(
        pl.kernel,
        out_type=jax.ShapeDtypeStruct((n, ddim), jnp.float32),
        mesh=mesh,
        scratch_types=[
            pltpu.VMEM((bpw,), jnp.int32),
            pltpu.VMEM((bpw, ddim), jnp.float32),
            pltpu.SemaphoreType.DMA,
        ],
    )
    def _sc_gather(w_hbm, idx_hbm, out_hbm, idx_v, rows_v, sem):
        wid = lax.axis_index("s") * ncores + lax.axis_index("c")
        base = wid * bpw
        pltpu.sync_copy(idx_hbm.at[pl.ds(base, bpw)], idx_v)
        pltpu.async_copy(w_hbm.at[idx_v], rows_v, sem).wait()
        pltpu.sync_copy(rows_v, out_hbm.at[pl.ds(base, bpw)])

    return _sc_gather


def _pipeline(inputs_l, weight, roff, n_total, axis_name):
    x = jnp.transpose(inputs_l, (0, 2, 3, 1))
    b, h, w, dd = x.shape
    flat = x.reshape(-1, dd)
    k = weight.shape[0]

    xn = jnp.sum(flat ** 2, axis=1, keepdims=True)
    wn = jnp.sum(weight ** 2, axis=1)[None, :]

    ent, idx = _dist_call(roff, xn, wn, flat, weight)
    quant = _sc_gather_build(flat.shape[0], dd)(weight, idx.reshape(-1))
    enc, cnt, esum = _onehot_call(idx, ent, k)

    if axis_name is not None:
        cnt = lax.psum(cnt, axis_name)
        esum = lax.psum(esum, axis_name)
    loss11, perp11 = _finalize_call(cnt, esum, n_total)

    q_l = jnp.transpose(quant.reshape(b, h, w, dd), (0, 3, 1, 2))
    return loss11, q_l, perp11, enc


def kernel(inputs, weight):
    n_total = inputs.shape[0] * inputs.shape[2] * inputs.shape[3]
    roff = jnp.zeros((1,), jnp.int32)
    loss11, q_out, perp11, enc = _pipeline(
        inputs, weight, roff, n_total, None)
    return (loss11[0, 0], q_out, perp11[0, 0], enc)
